# baseline (speedup 1.0000x reference)
import functools

import jax
import jax.numpy as jnp
from jax import lax
from jax.experimental import pallas as pl
from jax.experimental.pallas import tpu as pltpu
from jax.experimental.pallas import tpu_sc as plsc

N = 10000
NPAD = 10240
E = 320000
D1 = 128
D2P = 48
NCLASS = 40

NWORK = 32
CHUNK = 128
NCHT = 160
EPAD = NCHT * 16 * CHUNK
PH = 32
NCH0_AGG = 40
EPACK = EPAD + 2 * PH * CHUNK
ROWS_PT = NPAD // 16

def _fill_zeros_2d(buf, nrow, ncol):
    kper = ncol // 16

    def body(t, _):
        i = t // kper
        k = t % kper
        buf[i, pl.ds(k * 16, 16)] = jnp.zeros((16,), jnp.float32)
        return 0

    lax.fori_loop(0, nrow * kper, body, 0)


@functools.lru_cache(maxsize=None)
def _make_agg(dcols, has_rows, nch0=NCHT // 2):
    nch1 = NCHT - nch0

    def body(pk_hbm, hs_hbm, out_hbm,
             pk_v, siA, siB, diA, diB, bufA, bufB, acc, semA, semB, semP):
        c = lax.axis_index("c")
        s = lax.axis_index("s")
        nch = jnp.where(c == 0, nch0, nch1)
        start_chunk = jnp.where(c == 0, s * nch0, 16 * nch0 + s * nch1)

        _fill_zeros_2d(bufA, CHUNK, dcols)
        lo = s * ROWS_PT

        def zacc(m, _):
            pltpu.sync_copy(bufA, acc.at[pl.ds(lo + m * CHUNK, CHUNK)])
            return 0

        lax.fori_loop(0, ROWS_PT // CHUNK, zacc, 0)

        pltpu.sync_copy(pk_hbm.at[pl.ds(start_chunk * CHUNK, 2 * PH * CHUNK)],
                        pk_v)

        def pk_issue(phase):
            pltpu.async_copy(
                pk_hbm.at[pl.ds((start_chunk + phase * PH) * CHUNK,
                                PH * CHUNK)],
                pk_v.at[pl.ds((phase % 2) * PH * CHUNK, PH * CHUNK)],
                semP)

        def pk_wait():
            pltpu.make_async_copy(
                pk_hbm.at[pl.ds(0, PH * CHUNK)],
                pk_v.at[pl.ds(0, PH * CHUNK)], semP).wait()

        def pk_boundary(b):
            @pl.when(b >= 2 * PH)
            def _():
                pk_wait()

            @pl.when(b + PH < nch)
            def _():
                pk_issue(b // PH + 1)

        def unpack(j, sbuf, dbuf):
            roff = jnp.bitwise_and(j, 2 * PH - 1) * CHUNK

            def u(t, _):
                v = pk_v[pl.ds(roff + t * 16, 16)]
                if sbuf is not None:
                    sbuf[pl.ds(t * 16, 16)] = jnp.bitwise_and(v, 0xFFFF)
                dbuf[pl.ds(t * 16, 16)] = jnp.right_shift(v, 16)
                return 0

            lax.fori_loop(0, CHUNK // 16, u, 0)

        if not has_rows:
            def fill1(t, _):
                bufB[t, pl.ds(0, 16)] = jnp.full((16,), 1.0, jnp.float32)
                return 0

            lax.fori_loop(0, CHUNK, fill1, 0)
            plsc.subcore_barrier()

            def chunk(j, _):
                @pl.when((jnp.bitwise_and(j, PH - 1) == 0) & (j > 0))
                def _():
                    pk_boundary(j)

                unpack(j, None, diA)
                pltpu.sync_copy(bufB, acc.at[diA], add=True)
                return 0

            lax.fori_loop(0, nch, chunk, 0)
        else:
            plsc.subcore_barrier()

            def g_start(sbuf, buf, sem):
                pltpu.async_copy(hs_hbm.at[sbuf], buf, sem)

            def g_wait(sbuf, buf, sem):
                pltpu.make_async_copy(hs_hbm.at[sbuf], buf, sem).wait()

            @pl.when(nch > 0)
            def _():
                unpack(0, siA, diA)
                g_start(siA, bufA, semA)

            def pair(jj, _):
                j = jj * 2
                unpack(j + 1, siB, diB)
                g_start(siB, bufB, semB)
                g_wait(siA, bufA, semA)
                pltpu.sync_copy(bufA, acc.at[diA], add=True)

                @pl.when(jj < nch // 2 - 1)
                def _():
                    @pl.when(jnp.bitwise_and(j + 2, PH - 1) == 0)
                    def _():
                        pk_boundary(j + 2)

                    unpack(j + 2, siA, diA)
                    g_start(siA, bufA, semA)

                g_wait(siB, bufB, semB)
                pltpu.sync_copy(bufB, acc.at[diB], add=True)
                return 0

            lax.fori_loop(0, nch // 2, pair, 0)

        plsc.subcore_barrier()
        pltpu.sync_copy(acc.at[pl.ds(lo, ROWS_PT)],
                        out_hbm.at[c, pl.ds(lo, ROWS_PT)])

    return pl.kernel(
        body,
        mesh=plsc.VectorSubcoreMesh(core_axis_name="c", subcore_axis_name="s"),
        compiler_params=pltpu.CompilerParams(
            use_tc_tiling_on_sc=(dcols % 128 == 0)),
        out_type=jax.ShapeDtypeStruct((2, NPAD, dcols), jnp.float32),
        scratch_types=[
            pltpu.VMEM((2 * PH * CHUNK,), jnp.int32),
            pltpu.VMEM((CHUNK,), jnp.int32),
            pltpu.VMEM((CHUNK,), jnp.int32),
            pltpu.VMEM((CHUNK,), jnp.int32),
            pltpu.VMEM((CHUNK,), jnp.int32),
            pltpu.VMEM((CHUNK, dcols), jnp.float32),
            pltpu.VMEM((CHUNK, dcols), jnp.float32),
            pltpu.VMEM_SHARED((NPAD, dcols), jnp.float32),
            pltpu.SemaphoreType.DMA,
            pltpu.SemaphoreType.DMA,
            pltpu.SemaphoreType.DMA,
        ],
    )


def _tc1_body(x_ref, w_ref, degp_ref, hs_ref, dis_ref):
    deg = degp_ref[:, 0:1] + degp_ref[:, 1:2] + 1.0
    dis = lax.rsqrt(deg)
    h = jnp.dot(x_ref[:, :], w_ref[:, :], preferred_element_type=jnp.float32)
    hs_ref[:, :] = h * dis
    dis_ref[:, :] = dis


def _tc2_body(a_ref, hs1_ref, dis_ref, w2_ref, b1_ref, hs2_ref):
    dis = dis_ref[:, :]
    h = (a_ref[0] + a_ref[1] + hs1_ref[:, :]) * dis + b1_ref[:, :]
    h = jnp.maximum(h, 0.0)
    hs2_ref[:, :] = jnp.dot(h, w2_ref[:, :],
                            preferred_element_type=jnp.float32) * dis


def _tc3_body(a_ref, hs2_ref, dis_ref, b2_ref, out_ref):
    out_ref[:, :] = ((a_ref[0] + a_ref[1] + hs2_ref[:, :]) * dis_ref[:, :]
                     + b2_ref[:, :])


_RB = 1024
_GRID = NPAD // _RB


def _tc1(xpad, W1, degpT):
    return pl.pallas_call(
        _tc1_body,
        grid=(_GRID,),
        in_specs=[
            pl.BlockSpec((_RB, D1), lambda i: (i, 0)),
            pl.BlockSpec((D1, D1), lambda i: (0, 0)),
            pl.BlockSpec((_RB, 2), lambda i: (i, 0)),
        ],
        out_specs=[
            pl.BlockSpec((_RB, D1), lambda i: (i, 0)),
            pl.BlockSpec((_RB, 1), lambda i: (i, 0)),
        ],
        out_shape=[
            jax.ShapeDtypeStruct((NPAD, D1), jnp.float32),
            jax.ShapeDtypeStruct((NPAD, 1), jnp.float32),
        ],
    )(xpad, W1, degpT)


def _tc2(aggp1, hs1, dis, W2p, b1):
    return pl.pallas_call(
        _tc2_body,
        grid=(_GRID,),
        in_specs=[
            pl.BlockSpec((2, _RB, D1), lambda i: (0, i, 0)),
            pl.BlockSpec((_RB, D1), lambda i: (i, 0)),
            pl.BlockSpec((_RB, 1), lambda i: (i, 0)),
            pl.BlockSpec((D1, D2P), lambda i: (0, 0)),
            pl.BlockSpec((1, D1), lambda i: (0, 0)),
        ],
        out_specs=pl.BlockSpec((_RB, D2P), lambda i: (i, 0)),
        out_shape=jax.ShapeDtypeStruct((NPAD, D2P), jnp.float32),
    )(aggp1, hs1, dis, W2p, b1)


def _tc3(aggp2, hs2, dis, b2p):
    return pl.pallas_call(
        _tc3_body,
        grid=(_GRID,),
        in_specs=[
            pl.BlockSpec((2, _RB, D2P), lambda i: (0, i, 0)),
            pl.BlockSpec((_RB, D2P), lambda i: (i, 0)),
            pl.BlockSpec((_RB, 1), lambda i: (i, 0)),
            pl.BlockSpec((1, D2P), lambda i: (0, 0)),
        ],
        out_specs=pl.BlockSpec((_RB, D2P), lambda i: (i, 0)),
        out_shape=jax.ShapeDtypeStruct((NPAD, D2P), jnp.float32),
    )(aggp2, hs2, dis, b2p)


def kernel(x, edge_index, W1, b1, W2, b2):
    src = edge_index[0]
    dst = edge_index[1]
    packed = jnp.left_shift(dst, 16) | src
    pad = jnp.full((EPACK - E,), (N << 16) | N, jnp.int32)
    packed = jnp.concatenate([packed, pad])
    xpad = jnp.zeros((NPAD, D1), jnp.float32).at[:N].set(x)
    W2p = jnp.zeros((D1, D2P), jnp.float32).at[:, :NCLASS].set(W2)
    b2p = jnp.zeros((1, D2P), jnp.float32).at[0, :NCLASS].set(b2)
    b1r = b1.reshape(1, D1)

    degp = _make_agg(16, False)(packed, jnp.zeros((NPAD, 16), jnp.float32))
    degpT = degp[:, :, 0].T
    hs1, dis = _tc1(xpad, W1, degpT)
    aggp1 = _make_agg(D1, True, NCH0_AGG)(packed, hs1)
    hs2 = _tc2(aggp1, hs1, dis, W2p, b1r)
    aggp2 = _make_agg(D2P, True, NCH0_AGG)(packed, hs2)
    outp = _tc3(aggp2, hs2, dis, b2p)
    return outp[:N, :NCLASS]


# speedup vs baseline: 1.1225x; 1.1225x over previous
import functools

import jax
import jax.numpy as jnp
from jax import lax
from jax.experimental import pallas as pl
from jax.experimental.pallas import tpu as pltpu
from jax.experimental.pallas import tpu_sc as plsc

N = 10000
NPAD = 10240
E = 320000
D1 = 128
D2P = 48
NCLASS = 40

NWORK = 32
CHUNK = 128
NCHT = 160
EPAD = NCHT * 16 * CHUNK
PH = 32
NCH0_AGG = 136
EPACK = EPAD + 2 * PH * CHUNK
ROWS_PT = NPAD // 16

def _fill_zeros_2d(buf, nrow, ncol):
    kper = ncol // 16

    def body(t, _):
        i = t // kper
        k = t % kper
        buf[i, pl.ds(k * 16, 16)] = jnp.zeros((16,), jnp.float32)
        return 0

    lax.fori_loop(0, nrow * kper, body, 0)


@functools.lru_cache(maxsize=None)
def _make_agg(dcols, has_rows, nch0=NCHT // 2):
    nch1 = NCHT - nch0

    def body(pk_hbm, hs_hbm, out_hbm,
             pk_v, siA, siB, diA, diB, bufA, bufB, acc, semA, semB, semP):
        c = lax.axis_index("c")
        s = lax.axis_index("s")
        nch = jnp.where(c == 0, nch0, nch1)
        start_chunk = jnp.where(c == 0, s * nch0, 16 * nch0 + s * nch1)

        _fill_zeros_2d(bufA, CHUNK, dcols)
        lo = s * ROWS_PT

        def zacc(m, _):
            pltpu.sync_copy(bufA, acc.at[pl.ds(lo + m * CHUNK, CHUNK)])
            return 0

        lax.fori_loop(0, ROWS_PT // CHUNK, zacc, 0)

        pltpu.sync_copy(pk_hbm.at[pl.ds(start_chunk * CHUNK, 2 * PH * CHUNK)],
                        pk_v)

        def pk_issue(phase):
            pltpu.async_copy(
                pk_hbm.at[pl.ds((start_chunk + phase * PH) * CHUNK,
                                PH * CHUNK)],
                pk_v.at[pl.ds((phase % 2) * PH * CHUNK, PH * CHUNK)],
                semP)

        def pk_wait():
            pltpu.make_async_copy(
                pk_hbm.at[pl.ds(0, PH * CHUNK)],
                pk_v.at[pl.ds(0, PH * CHUNK)], semP).wait()

        def pk_boundary(b):
            @pl.when(b >= 2 * PH)
            def _():
                pk_wait()

            @pl.when(b + PH < nch)
            def _():
                pk_issue(b // PH + 1)

        def unpack(j, sbuf, dbuf):
            roff = jnp.bitwise_and(j, 2 * PH - 1) * CHUNK

            def u(t, _):
                v = pk_v[pl.ds(roff + t * 16, 16)]
                if sbuf is not None:
                    sbuf[pl.ds(t * 16, 16)] = jnp.bitwise_and(v, 0xFFFF)
                dbuf[pl.ds(t * 16, 16)] = jnp.right_shift(v, 16)
                return 0

            lax.fori_loop(0, CHUNK // 16, u, 0)

        if not has_rows:
            def fill1(t, _):
                bufB[t, pl.ds(0, 16)] = jnp.full((16,), 1.0, jnp.float32)
                return 0

            lax.fori_loop(0, CHUNK, fill1, 0)
            plsc.subcore_barrier()

            def chunk(j, _):
                @pl.when((jnp.bitwise_and(j, PH - 1) == 0) & (j > 0))
                def _():
                    pk_boundary(j)

                unpack(j, None, diA)
                pltpu.sync_copy(bufB, acc.at[diA], add=True)
                return 0

            lax.fori_loop(0, nch, chunk, 0)
        else:
            plsc.subcore_barrier()

            def g_start(sbuf, buf, sem):
                pltpu.async_copy(hs_hbm.at[sbuf], buf, sem)

            def g_wait(sbuf, buf, sem):
                pltpu.make_async_copy(hs_hbm.at[sbuf], buf, sem).wait()

            @pl.when(nch > 0)
            def _():
                unpack(0, siA, diA)
                g_start(siA, bufA, semA)

            def pair(jj, _):
                j = jj * 2
                unpack(j + 1, siB, diB)
                g_start(siB, bufB, semB)
                g_wait(siA, bufA, semA)
                pltpu.sync_copy(bufA, acc.at[diA], add=True)

                @pl.when(jj < nch // 2 - 1)
                def _():
                    @pl.when(jnp.bitwise_and(j + 2, PH - 1) == 0)
                    def _():
                        pk_boundary(j + 2)

                    unpack(j + 2, siA, diA)
                    g_start(siA, bufA, semA)

                g_wait(siB, bufB, semB)
                pltpu.sync_copy(bufB, acc.at[diB], add=True)
                return 0

            lax.fori_loop(0, nch // 2, pair, 0)

        plsc.subcore_barrier()
        pltpu.sync_copy(acc.at[pl.ds(lo, ROWS_PT)],
                        out_hbm.at[c, pl.ds(lo, ROWS_PT)])

    return pl.kernel(
        body,
        mesh=plsc.VectorSubcoreMesh(core_axis_name="c", subcore_axis_name="s"),
        compiler_params=pltpu.CompilerParams(
            use_tc_tiling_on_sc=(dcols % 128 == 0)),
        out_type=jax.ShapeDtypeStruct((2, NPAD, dcols), jnp.float32),
        scratch_types=[
            pltpu.VMEM((2 * PH * CHUNK,), jnp.int32),
            pltpu.VMEM((CHUNK,), jnp.int32),
            pltpu.VMEM((CHUNK,), jnp.int32),
            pltpu.VMEM((CHUNK,), jnp.int32),
            pltpu.VMEM((CHUNK,), jnp.int32),
            pltpu.VMEM((CHUNK, dcols), jnp.float32),
            pltpu.VMEM((CHUNK, dcols), jnp.float32),
            pltpu.VMEM_SHARED((NPAD, dcols), jnp.float32),
            pltpu.SemaphoreType.DMA,
            pltpu.SemaphoreType.DMA,
            pltpu.SemaphoreType.DMA,
        ],
    )


def _tc1_body(x_ref, w_ref, degp_ref, hs_ref, dis_ref):
    deg = degp_ref[:, 0:1] + degp_ref[:, 1:2] + 1.0
    dis = lax.rsqrt(deg)
    h = jnp.dot(x_ref[:, :], w_ref[:, :], preferred_element_type=jnp.float32)
    hs_ref[:, :] = h * dis
    dis_ref[:, :] = dis


def _tc2_body(a_ref, hs1_ref, dis_ref, w2_ref, b1_ref, hs2_ref):
    dis = dis_ref[:, :]
    h = (a_ref[0] + a_ref[1] + hs1_ref[:, :]) * dis + b1_ref[:, :]
    h = jnp.maximum(h, 0.0)
    hs2_ref[:, :] = jnp.dot(h, w2_ref[:, :],
                            preferred_element_type=jnp.float32) * dis


def _tc3_body(a_ref, hs2_ref, dis_ref, b2_ref, out_ref):
    out_ref[:, :] = ((a_ref[0] + a_ref[1] + hs2_ref[:, :]) * dis_ref[:, :]
                     + b2_ref[:, :])


_RB = 1024
_GRID = NPAD // _RB


def _tc1(xpad, W1, degpT):
    return pl.pallas_call(
        _tc1_body,
        grid=(_GRID,),
        in_specs=[
            pl.BlockSpec((_RB, D1), lambda i: (i, 0)),
            pl.BlockSpec((D1, D1), lambda i: (0, 0)),
            pl.BlockSpec((_RB, 2), lambda i: (i, 0)),
        ],
        out_specs=[
            pl.BlockSpec((_RB, D1), lambda i: (i, 0)),
            pl.BlockSpec((_RB, 1), lambda i: (i, 0)),
        ],
        out_shape=[
            jax.ShapeDtypeStruct((NPAD, D1), jnp.float32),
            jax.ShapeDtypeStruct((NPAD, 1), jnp.float32),
        ],
    )(xpad, W1, degpT)


def _tc2(aggp1, hs1, dis, W2p, b1):
    return pl.pallas_call(
        _tc2_body,
        grid=(_GRID,),
        in_specs=[
            pl.BlockSpec((2, _RB, D1), lambda i: (0, i, 0)),
            pl.BlockSpec((_RB, D1), lambda i: (i, 0)),
            pl.BlockSpec((_RB, 1), lambda i: (i, 0)),
            pl.BlockSpec((D1, D2P), lambda i: (0, 0)),
            pl.BlockSpec((1, D1), lambda i: (0, 0)),
        ],
        out_specs=pl.BlockSpec((_RB, D2P), lambda i: (i, 0)),
        out_shape=jax.ShapeDtypeStruct((NPAD, D2P), jnp.float32),
    )(aggp1, hs1, dis, W2p, b1)


def _tc3(aggp2, hs2, dis, b2p):
    return pl.pallas_call(
        _tc3_body,
        grid=(_GRID,),
        in_specs=[
            pl.BlockSpec((2, _RB, D2P), lambda i: (0, i, 0)),
            pl.BlockSpec((_RB, D2P), lambda i: (i, 0)),
            pl.BlockSpec((_RB, 1), lambda i: (i, 0)),
            pl.BlockSpec((1, D2P), lambda i: (0, 0)),
        ],
        out_specs=pl.BlockSpec((_RB, D2P), lambda i: (i, 0)),
        out_shape=jax.ShapeDtypeStruct((NPAD, D2P), jnp.float32),
    )(aggp2, hs2, dis, b2p)


def kernel(x, edge_index, W1, b1, W2, b2):
    src = edge_index[0]
    dst = edge_index[1]
    packed = jnp.left_shift(dst, 16) | src
    pad = jnp.full((EPACK - E,), (N << 16) | N, jnp.int32)
    packed = jnp.concatenate([packed, pad])
    xpad = jnp.zeros((NPAD, D1), jnp.float32).at[:N].set(x)
    W2p = jnp.zeros((D1, D2P), jnp.float32).at[:, :NCLASS].set(W2)
    b2p = jnp.zeros((1, D2P), jnp.float32).at[0, :NCLASS].set(b2)
    b1r = b1.reshape(1, D1)

    degp = _make_agg(16, False)(packed, jnp.zeros((NPAD, 16), jnp.float32))
    degpT = degp[:, :, 0].T
    hs1, dis = _tc1(xpad, W1, degpT)
    aggp1 = _make_agg(D1, True, NCH0_AGG)(packed, hs1)
    hs2 = _tc2(aggp1, hs1, dis, W2p, b1r)
    aggp2 = _make_agg(D2P, True, NCH0_AGG)(packed, hs2)
    outp = _tc3(aggp2, hs2, dis, b2p)
    return outp[:N, :NCLASS]


# speedup vs baseline: 1.1951x; 1.0647x over previous
import functools

import jax
import jax.numpy as jnp
from jax import lax
from jax.experimental import pallas as pl
from jax.experimental.pallas import tpu as pltpu
from jax.experimental.pallas import tpu_sc as plsc

N = 10000
NPAD = 10240
E = 320000
D1 = 128
D2P = 48
NCLASS = 40

NWORK = 32
CHUNK = 128
NCHT = 160
EPAD = NCHT * 16 * CHUNK
PH = 32
NCH0_AGG = 148
EPACK = EPAD + 2 * PH * CHUNK
ROWS_PT = NPAD // 16

def _fill_zeros_2d(buf, nrow, ncol):
    kper = ncol // 16

    def body(t, _):
        i = t // kper
        k = t % kper
        buf[i, pl.ds(k * 16, 16)] = jnp.zeros((16,), jnp.float32)
        return 0

    lax.fori_loop(0, nrow * kper, body, 0)


@functools.lru_cache(maxsize=None)
def _make_agg(dcols, has_rows, nch0=NCHT // 2):
    nch1 = NCHT - nch0

    def body(pk_hbm, hs_hbm, out_hbm,
             pk_v, siA, siB, diA, diB, bufA, bufB, acc, semA, semB, semP):
        c = lax.axis_index("c")
        s = lax.axis_index("s")
        nch = jnp.where(c == 0, nch0, nch1)
        start_chunk = jnp.where(c == 0, s * nch0, 16 * nch0 + s * nch1)

        _fill_zeros_2d(bufA, CHUNK, dcols)
        lo = s * ROWS_PT

        def zacc(m, _):
            pltpu.sync_copy(bufA, acc.at[pl.ds(lo + m * CHUNK, CHUNK)])
            return 0

        lax.fori_loop(0, ROWS_PT // CHUNK, zacc, 0)

        pltpu.sync_copy(pk_hbm.at[pl.ds(start_chunk * CHUNK, 2 * PH * CHUNK)],
                        pk_v)

        def pk_issue(phase):
            pltpu.async_copy(
                pk_hbm.at[pl.ds((start_chunk + phase * PH) * CHUNK,
                                PH * CHUNK)],
                pk_v.at[pl.ds((phase % 2) * PH * CHUNK, PH * CHUNK)],
                semP)

        def pk_wait():
            pltpu.make_async_copy(
                pk_hbm.at[pl.ds(0, PH * CHUNK)],
                pk_v.at[pl.ds(0, PH * CHUNK)], semP).wait()

        def pk_boundary(b):
            @pl.when(b >= 2 * PH)
            def _():
                pk_wait()

            @pl.when(b + PH < nch)
            def _():
                pk_issue(b // PH + 1)

        def unpack(j, sbuf, dbuf):
            roff = jnp.bitwise_and(j, 2 * PH - 1) * CHUNK

            def u(t, _):
                v = pk_v[pl.ds(roff + t * 16, 16)]
                if sbuf is not None:
                    sbuf[pl.ds(t * 16, 16)] = jnp.bitwise_and(v, 0xFFFF)
                dbuf[pl.ds(t * 16, 16)] = jnp.right_shift(v, 16)
                return 0

            lax.fori_loop(0, CHUNK // 16, u, 0)

        if not has_rows:
            def fill1(t, _):
                bufB[t, pl.ds(0, 16)] = jnp.full((16,), 1.0, jnp.float32)
                return 0

            lax.fori_loop(0, CHUNK, fill1, 0)
            plsc.subcore_barrier()

            def chunk(j, _):
                @pl.when((jnp.bitwise_and(j, PH - 1) == 0) & (j > 0))
                def _():
                    pk_boundary(j)

                unpack(j, None, diA)
                pltpu.sync_copy(bufB, acc.at[diA], add=True)
                return 0

            lax.fori_loop(0, nch, chunk, 0)
        else:
            plsc.subcore_barrier()

            def g_start(sbuf, buf, sem):
                pltpu.async_copy(hs_hbm.at[sbuf], buf, sem)

            def g_wait(sbuf, buf, sem):
                pltpu.make_async_copy(hs_hbm.at[sbuf], buf, sem).wait()

            @pl.when(nch > 0)
            def _():
                unpack(0, siA, diA)
                g_start(siA, bufA, semA)

            def pair(jj, _):
                j = jj * 2
                unpack(j + 1, siB, diB)
                g_start(siB, bufB, semB)
                g_wait(siA, bufA, semA)
                pltpu.sync_copy(bufA, acc.at[diA], add=True)

                @pl.when(jj < nch // 2 - 1)
                def _():
                    @pl.when(jnp.bitwise_and(j + 2, PH - 1) == 0)
                    def _():
                        pk_boundary(j + 2)

                    unpack(j + 2, siA, diA)
                    g_start(siA, bufA, semA)

                g_wait(siB, bufB, semB)
                pltpu.sync_copy(bufB, acc.at[diB], add=True)
                return 0

            lax.fori_loop(0, nch // 2, pair, 0)

        plsc.subcore_barrier()
        pltpu.sync_copy(acc.at[pl.ds(lo, ROWS_PT)],
                        out_hbm.at[c, pl.ds(lo, ROWS_PT)])

    return pl.kernel(
        body,
        mesh=plsc.VectorSubcoreMesh(core_axis_name="c", subcore_axis_name="s"),
        compiler_params=pltpu.CompilerParams(
            use_tc_tiling_on_sc=(dcols % 128 == 0)),
        out_type=jax.ShapeDtypeStruct((2, NPAD, dcols), jnp.float32),
        scratch_types=[
            pltpu.VMEM((2 * PH * CHUNK,), jnp.int32),
            pltpu.VMEM((CHUNK,), jnp.int32),
            pltpu.VMEM((CHUNK,), jnp.int32),
            pltpu.VMEM((CHUNK,), jnp.int32),
            pltpu.VMEM((CHUNK,), jnp.int32),
            pltpu.VMEM((CHUNK, dcols), jnp.float32),
            pltpu.VMEM((CHUNK, dcols), jnp.float32),
            pltpu.VMEM_SHARED((NPAD, dcols), jnp.float32),
            pltpu.SemaphoreType.DMA,
            pltpu.SemaphoreType.DMA,
            pltpu.SemaphoreType.DMA,
        ],
    )


def _tc1_body(x_ref, w_ref, degp_ref, hs_ref, dis_ref):
    deg = degp_ref[:, 0:1] + degp_ref[:, 1:2] + 1.0
    dis = lax.rsqrt(deg)
    h = jnp.dot(x_ref[:, :], w_ref[:, :], preferred_element_type=jnp.float32)
    hs_ref[:, :] = h * dis
    dis_ref[:, :] = dis


def _tc2_body(a_ref, hs1_ref, dis_ref, w2_ref, b1_ref, hs2_ref):
    dis = dis_ref[:, :]
    h = (a_ref[0] + a_ref[1] + hs1_ref[:, :]) * dis + b1_ref[:, :]
    h = jnp.maximum(h, 0.0)
    hs2_ref[:, :] = jnp.dot(h, w2_ref[:, :],
                            preferred_element_type=jnp.float32) * dis


def _tc3_body(a_ref, hs2_ref, dis_ref, b2_ref, out_ref):
    out_ref[:, :] = ((a_ref[0] + a_ref[1] + hs2_ref[:, :]) * dis_ref[:, :]
                     + b2_ref[:, :])


_RB = 1024
_GRID = NPAD // _RB


def _tc1(xpad, W1, degpT):
    return pl.pallas_call(
        _tc1_body,
        grid=(_GRID,),
        in_specs=[
            pl.BlockSpec((_RB, D1), lambda i: (i, 0)),
            pl.BlockSpec((D1, D1), lambda i: (0, 0)),
            pl.BlockSpec((_RB, 2), lambda i: (i, 0)),
        ],
        out_specs=[
            pl.BlockSpec((_RB, D1), lambda i: (i, 0)),
            pl.BlockSpec((_RB, 1), lambda i: (i, 0)),
        ],
        out_shape=[
            jax.ShapeDtypeStruct((NPAD, D1), jnp.float32),
            jax.ShapeDtypeStruct((NPAD, 1), jnp.float32),
        ],
    )(xpad, W1, degpT)


def _tc2(aggp1, hs1, dis, W2p, b1):
    return pl.pallas_call(
        _tc2_body,
        grid=(_GRID,),
        in_specs=[
            pl.BlockSpec((2, _RB, D1), lambda i: (0, i, 0)),
            pl.BlockSpec((_RB, D1), lambda i: (i, 0)),
            pl.BlockSpec((_RB, 1), lambda i: (i, 0)),
            pl.BlockSpec((D1, D2P), lambda i: (0, 0)),
            pl.BlockSpec((1, D1), lambda i: (0, 0)),
        ],
        out_specs=pl.BlockSpec((_RB, D2P), lambda i: (i, 0)),
        out_shape=jax.ShapeDtypeStruct((NPAD, D2P), jnp.float32),
    )(aggp1, hs1, dis, W2p, b1)


def _tc3(aggp2, hs2, dis, b2p):
    return pl.pallas_call(
        _tc3_body,
        grid=(_GRID,),
        in_specs=[
            pl.BlockSpec((2, _RB, D2P), lambda i: (0, i, 0)),
            pl.BlockSpec((_RB, D2P), lambda i: (i, 0)),
            pl.BlockSpec((_RB, 1), lambda i: (i, 0)),
            pl.BlockSpec((1, D2P), lambda i: (0, 0)),
        ],
        out_specs=pl.BlockSpec((_RB, D2P), lambda i: (i, 0)),
        out_shape=jax.ShapeDtypeStruct((NPAD, D2P), jnp.float32),
    )(aggp2, hs2, dis, b2p)


def kernel(x, edge_index, W1, b1, W2, b2):
    src = edge_index[0]
    dst = edge_index[1]
    packed = jnp.left_shift(dst, 16) | src
    pad = jnp.full((EPACK - E,), (N << 16) | N, jnp.int32)
    packed = jnp.concatenate([packed, pad])
    xpad = jnp.zeros((NPAD, D1), jnp.float32).at[:N].set(x)
    W2p = jnp.zeros((D1, D2P), jnp.float32).at[:, :NCLASS].set(W2)
    b2p = jnp.zeros((1, D2P), jnp.float32).at[0, :NCLASS].set(b2)
    b1r = b1.reshape(1, D1)

    degp = _make_agg(16, False)(packed, jnp.zeros((NPAD, 16), jnp.float32))
    degpT = degp[:, :, 0].T
    hs1, dis = _tc1(xpad, W1, degpT)
    aggp1 = _make_agg(D1, True, NCH0_AGG)(packed, hs1)
    hs2 = _tc2(aggp1, hs1, dis, W2p, b1r)
    aggp2 = _make_agg(D2P, True, NCH0_AGG)(packed, hs2)
    outp = _tc3(aggp2, hs2, dis, b2p)
    return outp[:N, :NCLASS]
